# baseline (speedup 1.0000x reference)
import functools

import jax
import jax.numpy as jnp
from jax import lax
from jax.experimental import pallas as pl
from jax.experimental.pallas import tpu as pltpu
from jax.experimental.pallas import tpu_sc as plsc

MD = 4096
NE = 64
TOKENS = 8192
KK = 2
TB = 256
NBLK = TOKENS // TB
W_LB = 0.01
W_Z = 0.001

N = TOKENS * KK
NC = 2
NS = 16
NW = NC * NS
CH = N // NW
SUB = 128
NSUB = CH // SUB


def _router_block(x_ref, w_ref, keys_ref, cw_ref, ranks_ref, hist_ref,
                  offs_ref, laux_ref, H, Psum, Cnt, Z):
    b = pl.program_id(0)

    @pl.when(b == 0)
    def _init():
        H[...] = jnp.zeros_like(H)
        Psum[...] = jnp.zeros_like(Psum)
        Cnt[...] = jnp.zeros_like(Cnt)
        Z[...] = jnp.zeros_like(Z)

    x = x_ref[...]
    w = w_ref[...]
    logits = lax.dot_general(x, w, (((1,), (1,)), ((), ())),
                             preferred_element_type=jnp.float32)
    m = jnp.max(logits, axis=1, keepdims=True)
    e = jnp.exp(logits - m)
    s = jnp.sum(e, axis=1, keepdims=True)
    probs = e / s

    li = lax.broadcasted_iota(jnp.int32, (TB, NE), 1)
    p0 = jnp.max(probs, axis=1, keepdims=True)
    i0 = jnp.min(jnp.where(probs == p0, li, NE), axis=1, keepdims=True)
    E0 = li == i0
    pm = jnp.where(E0, -1.0, probs)
    p1 = jnp.max(pm, axis=1, keepdims=True)
    i1 = jnp.min(jnp.where(pm == p1, li, NE), axis=1, keepdims=True)
    E1 = li == i1

    E0f = E0.astype(jnp.float32)
    E1f = E1.astype(jnp.float32)
    S = E0f + E1f

    r_i = lax.broadcasted_iota(jnp.int32, (TB, TB), 0)
    c_i = lax.broadcasted_iota(jnp.int32, (TB, TB), 1)
    ltri = (r_i > c_i).astype(jnp.float32)
    excl = lax.dot_general(ltri, S, (((1,), (0,)), ((), ())),
                           preferred_element_type=jnp.float32)
    Hrow = H[...]
    tot = excl + Hrow
    r0 = jnp.sum(tot * E0f, axis=1, keepdims=True)
    r1 = jnp.sum(tot * E1f, axis=1, keepdims=True)
    H[...] = Hrow + jnp.sum(S, axis=0, keepdims=True)

    ew = jnp.exp(p1 - p0)
    cw0 = 1.0 / (1.0 + ew)
    cw1 = ew * cw0

    col = lax.broadcasted_iota(jnp.int32, (TB, KK), 1)
    keys_ref[...] = jnp.where(col == 0, i0, i1)
    ranks_ref[...] = jnp.where(col == 0, r0, r1).astype(jnp.int32)
    cw_ref[...] = jnp.where(col == 0, cw0, cw1)
    hist_ref[...] = H[...].astype(jnp.int32)
    e_r = lax.broadcasted_iota(jnp.int32, (NE, NE), 0)
    e_c = lax.broadcasted_iota(jnp.int32, (NE, NE), 1)
    utri = (e_r < e_c).astype(jnp.float32)
    offs_ref[...] = lax.dot_general(H[...], utri, (((1,), (0,)), ((), ())),
                                    precision=lax.Precision.HIGHEST,
                                    preferred_element_type=jnp.float32
                                    ).astype(jnp.int32)

    Psum[...] += jnp.sum(probs, axis=0, keepdims=True)
    Cnt[...] += jnp.sum(E0f, axis=0, keepdims=True)
    lse = m + jnp.log(s)
    Z[...] += jnp.sum(lse * lse)

    @pl.when(b == NBLK - 1)
    def _finish():
        me_ce = (Psum[...] / TOKENS) * (Cnt[...] / TOKENS)
        lb = NE * jnp.sum(me_ce)
        z = Z[0, 0] / TOKENS
        laux_ref[...] = jnp.full((1, 1), W_LB * lb + W_Z * z, jnp.float32)


def _router(inputs, W):
    return pl.pallas_call(
        _router_block,
        grid=(NBLK,),
        in_specs=[
            pl.BlockSpec((TB, MD), lambda b: (b, 0)),
            pl.BlockSpec((NE, MD), lambda b: (0, 0)),
        ],
        out_specs=[
            pl.BlockSpec((TB, KK), lambda b: (b, 0)),
            pl.BlockSpec((TB, KK), lambda b: (b, 0)),
            pl.BlockSpec((TB, KK), lambda b: (b, 0)),
            pl.BlockSpec((1, NE), lambda b: (0, 0)),
            pl.BlockSpec((1, NE), lambda b: (0, 0)),
            pl.BlockSpec((1, 1), lambda b: (0, 0)),
        ],
        out_shape=[
            jax.ShapeDtypeStruct((TOKENS, KK), jnp.int32),
            jax.ShapeDtypeStruct((TOKENS, KK), jnp.float32),
            jax.ShapeDtypeStruct((TOKENS, KK), jnp.int32),
            jax.ShapeDtypeStruct((1, NE), jnp.int32),
            jax.ShapeDtypeStruct((1, NE), jnp.int32),
            jax.ShapeDtypeStruct((1, 1), jnp.float32),
        ],
        scratch_shapes=[
            pltpu.VMEM((1, NE), jnp.float32),
            pltpu.VMEM((1, NE), jnp.float32),
            pltpu.VMEM((1, NE), jnp.float32),
            pltpu.VMEM((1, 1), jnp.float32),
        ],
    )(inputs, W)


def _dispatch_body(keys_hbm, ranks_hbm, offs_hbm, rev_hbm, sort_hbm,
                   keys_v, ranks_v, off_v,
                   rv0, rv1, rv2, rv3, tv0, tv1, tv2, tv3, sem):
    wid = lax.axis_index("s") * NC + lax.axis_index("c")
    base = wid * CH
    c1 = pltpu.async_copy(keys_hbm.at[pl.ds(base, CH)], keys_v, sem)
    c2 = pltpu.async_copy(ranks_hbm.at[pl.ds(base, CH)], ranks_v, sem)
    c3 = pltpu.async_copy(offs_hbm, off_v, sem)
    c1.wait()
    c2.wait()
    c3.wait()

    rev_chunks = [rv0, rv1, rv2, rv3]
    tok_chunks = [tv0, tv1, tv2, tv3]
    lane = lax.iota(jnp.int32, 16)
    for v in range(CH // 16):
        k16 = keys_v[pl.ds(v * 16, 16)]
        r16 = ranks_v[pl.ds(v * 16, 16)]
        o16 = plsc.load_gather(off_v, [k16])
        rev16 = o16 + r16
        tok16 = lax.shift_right_logical(lane + (base + v * 16), 1)
        cid, coff = divmod(v * 16, SUB)
        rev_chunks[cid][pl.ds(coff, 16)] = rev16
        tok_chunks[cid][pl.ds(coff, 16)] = tok16

    outs = [
        pltpu.async_copy(rev_chunks[j], rev_hbm.at[pl.ds(base + j * SUB, SUB)],
                         sem)
        for j in range(NSUB)
    ]
    outs += [
        pltpu.async_copy(tok_chunks[j], sort_hbm.at[pl.ds(base + j * SUB, SUB)],
                         sem)
        for j in range(NSUB)
    ]
    for c in outs:
        c.wait()


@functools.cache
def _build_dispatch():
    return functools.partial(
        pl.kernel,
        mesh=plsc.VectorSubcoreMesh(core_axis_name="c", subcore_axis_name="s"),
        compiler_params=pltpu.CompilerParams(needs_layout_passes=False),
        out_type=[
            jax.ShapeDtypeStruct((N,), jnp.int32),
            jax.ShapeDtypeStruct((N,), jnp.int32),
        ],
        scratch_types=[
            pltpu.VMEM((CH,), jnp.int32),
            pltpu.VMEM((CH,), jnp.int32),
            pltpu.VMEM((NE,), jnp.int32),
            pltpu.VMEM((SUB,), jnp.int32),
            pltpu.VMEM((SUB,), jnp.int32),
            pltpu.VMEM((SUB,), jnp.int32),
            pltpu.VMEM((SUB,), jnp.int32),
            pltpu.VMEM((SUB,), jnp.int32),
            pltpu.VMEM((SUB,), jnp.int32),
            pltpu.VMEM((SUB,), jnp.int32),
            pltpu.VMEM((SUB,), jnp.int32),
            pltpu.SemaphoreType.DMA,
        ],
    )(_dispatch_body)


@jax.jit
def kernel(inputs, W):
    keys2, cw2, ranks2, hist2, offs2, laux = _router(inputs, W)
    keys = keys2.reshape(-1)
    ranks = ranks2.reshape(-1)
    hist = hist2.reshape(-1)
    rev, sort_ord = _build_dispatch()(keys, ranks, offs2.reshape(-1))
    return (sort_ord, rev, cw2.reshape(-1), hist, laux.reshape(()))


@functools.cache
def _build_probe():
    def _body(keys_hbm, ranks_hbm, offs_hbm, rev_hbm, sort_hbm, kv, sem):
        wid = lax.axis_index("s") * NC + lax.axis_index("c")
        base = wid * CH
        c1 = pltpu.async_copy(keys_hbm.at[pl.ds(base, CH)], kv, sem)
        c1.wait()
        c2 = pltpu.async_copy(kv, rev_hbm.at[pl.ds(base, CH)], sem)
        c3 = pltpu.async_copy(kv, sort_hbm.at[pl.ds(base, CH)], sem)
        c2.wait()
        c3.wait()

    return functools.partial(
        pl.kernel,
        mesh=plsc.VectorSubcoreMesh(core_axis_name="c", subcore_axis_name="s"),
        compiler_params=pltpu.CompilerParams(needs_layout_passes=False),
        out_type=[
            jax.ShapeDtypeStruct((N,), jnp.int32),
            jax.ShapeDtypeStruct((N,), jnp.int32),
        ],
        scratch_types=[
            pltpu.VMEM((CH,), jnp.int32),
            pltpu.SemaphoreType.DMA,
        ],
    )(_body)


# speedup vs baseline: 1.0129x; 1.0129x over previous
import functools

import jax
import jax.numpy as jnp
from jax import lax
from jax.experimental import pallas as pl
from jax.experimental.pallas import tpu as pltpu
from jax.experimental.pallas import tpu_sc as plsc

MD = 4096
NE = 64
TOKENS = 8192
KK = 2
TB = 256
NBLK = TOKENS // TB
W_LB = 0.01
W_Z = 0.001

N = TOKENS * KK
NC = 2
NS = 16
NW = NC * NS
CH = N // NW
SUB = 128
NSUB = CH // SUB


def _router_block(x_ref, w_ref, keys_ref, cw_ref, ranks_ref, hist_ref,
                  offs_ref, laux_ref, H, Psum, Cnt, Z):
    b = pl.program_id(0)

    @pl.when(b == 0)
    def _init():
        H[...] = jnp.zeros_like(H)
        Psum[...] = jnp.zeros_like(Psum)
        Cnt[...] = jnp.zeros_like(Cnt)
        Z[...] = jnp.zeros_like(Z)

    x = x_ref[...]
    w = w_ref[...]
    logits = lax.dot_general(x, w, (((1,), (1,)), ((), ())),
                             preferred_element_type=jnp.float32)
    m = jnp.max(logits, axis=1, keepdims=True)
    e = jnp.exp(logits - m)
    s = jnp.sum(e, axis=1, keepdims=True)
    probs = e / s

    li = lax.broadcasted_iota(jnp.int32, (TB, NE), 1)
    p0 = jnp.max(probs, axis=1, keepdims=True)
    i0 = jnp.min(jnp.where(probs == p0, li, NE), axis=1, keepdims=True)
    E0 = li == i0
    pm = jnp.where(E0, -1.0, probs)
    p1 = jnp.max(pm, axis=1, keepdims=True)
    i1 = jnp.min(jnp.where(pm == p1, li, NE), axis=1, keepdims=True)
    E1 = li == i1

    E0f = E0.astype(jnp.float32)
    E1f = E1.astype(jnp.float32)
    S = E0f + E1f

    r_i = lax.broadcasted_iota(jnp.int32, (TB, TB), 0)
    c_i = lax.broadcasted_iota(jnp.int32, (TB, TB), 1)
    ltri = (r_i > c_i).astype(jnp.float32)
    excl = lax.dot_general(ltri, S, (((1,), (0,)), ((), ())),
                           preferred_element_type=jnp.float32)
    Hrow = H[...]
    tot = excl + Hrow
    r0 = jnp.sum(tot * E0f, axis=1, keepdims=True)
    r1 = jnp.sum(tot * E1f, axis=1, keepdims=True)
    H[...] = Hrow + jnp.sum(S, axis=0, keepdims=True)

    ew = jnp.exp(p1 - p0)
    cw0 = 1.0 / (1.0 + ew)
    cw1 = ew * cw0

    col = lax.broadcasted_iota(jnp.int32, (TB, KK), 1)
    keys_ref[...] = jnp.where(col == 0, i0, i1)
    ranks_ref[...] = jnp.where(col == 0, r0, r1).astype(jnp.int32)
    cw_ref[...] = jnp.where(col == 0, cw0, cw1)
    hist_ref[...] = H[...].astype(jnp.int32)
    e_r = lax.broadcasted_iota(jnp.int32, (NE, NE), 0)
    e_c = lax.broadcasted_iota(jnp.int32, (NE, NE), 1)
    utri = (e_r < e_c).astype(jnp.float32)
    offs_ref[...] = lax.dot_general(H[...], utri, (((1,), (0,)), ((), ())),
                                    precision=lax.Precision.HIGHEST,
                                    preferred_element_type=jnp.float32
                                    ).astype(jnp.int32)

    Psum[...] += jnp.sum(probs, axis=0, keepdims=True)
    Cnt[...] += jnp.sum(E0f, axis=0, keepdims=True)
    lse = m + jnp.log(s)
    Z[...] += jnp.sum(lse * lse)

    @pl.when(b == NBLK - 1)
    def _finish():
        me_ce = (Psum[...] / TOKENS) * (Cnt[...] / TOKENS)
        lb = NE * jnp.sum(me_ce)
        z = Z[0, 0] / TOKENS
        laux_ref[...] = jnp.full((1, 1), W_LB * lb + W_Z * z, jnp.float32)


def _router(inputs, W):
    return pl.pallas_call(
        _router_block,
        grid=(NBLK,),
        in_specs=[
            pl.BlockSpec((TB, MD), lambda b: (b, 0)),
            pl.BlockSpec((NE, MD), lambda b: (0, 0)),
        ],
        out_specs=[
            pl.BlockSpec((TB, KK), lambda b: (b, 0)),
            pl.BlockSpec((TB, KK), lambda b: (b, 0)),
            pl.BlockSpec((TB, KK), lambda b: (b, 0)),
            pl.BlockSpec((1, NE), lambda b: (0, 0)),
            pl.BlockSpec((1, NE), lambda b: (0, 0)),
            pl.BlockSpec((1, 1), lambda b: (0, 0)),
        ],
        out_shape=[
            jax.ShapeDtypeStruct((TOKENS, KK), jnp.int32),
            jax.ShapeDtypeStruct((TOKENS, KK), jnp.float32),
            jax.ShapeDtypeStruct((TOKENS, KK), jnp.int32),
            jax.ShapeDtypeStruct((1, NE), jnp.int32),
            jax.ShapeDtypeStruct((1, NE), jnp.int32),
            jax.ShapeDtypeStruct((1, 1), jnp.float32),
        ],
        scratch_shapes=[
            pltpu.VMEM((1, NE), jnp.float32),
            pltpu.VMEM((1, NE), jnp.float32),
            pltpu.VMEM((1, NE), jnp.float32),
            pltpu.VMEM((1, 1), jnp.float32),
        ],
    )(inputs, W)


CH1 = N // NS
NSUB1 = CH1 // SUB


def _dispatch_body(keys_hbm, ranks_hbm, offs_hbm, rev_hbm, sort_hbm,
                   keys_v, ranks_v, off_v, revc, tokc, sort_sh, sem, sem2):
    sid = lax.axis_index("s")
    base = sid * CH1
    c1 = pltpu.async_copy(keys_hbm.at[pl.ds(base, CH1)], keys_v, sem)
    c2 = pltpu.async_copy(ranks_hbm.at[pl.ds(base, CH1)], ranks_v, sem)
    c3 = pltpu.async_copy(offs_hbm, off_v, sem)
    c1.wait()
    c2.wait()
    c3.wait()

    lane = lax.iota(jnp.int32, 16)
    for v in range(CH1 // 16):
        k16 = keys_v[pl.ds(v * 16, 16)]
        r16 = ranks_v[pl.ds(v * 16, 16)]
        o16 = plsc.load_gather(off_v, [k16])
        rev16 = o16 + r16
        tok16 = lax.shift_right_logical(lane + (base + v * 16), 1)
        cid, coff = divmod(v * 16, SUB)
        revc[cid][pl.ds(coff, 16)] = rev16
        tokc[cid][pl.ds(coff, 16)] = tok16

    outs = [
        pltpu.async_copy(revc[j], rev_hbm.at[pl.ds(base + j * SUB, SUB)], sem)
        for j in range(NSUB1)
    ]
    scats = [
        pltpu.async_copy(tokc[j], sort_sh.at[revc[j]], sem2)
        for j in range(NSUB1)
    ]
    for c in outs:
        c.wait()
    for c in scats:
        c.wait()
    plsc.subcore_barrier()
    pltpu.sync_copy(sort_sh.at[pl.ds(base, CH1)], keys_v)
    pltpu.sync_copy(keys_v, sort_hbm.at[pl.ds(base, CH1)])


@functools.cache
def _build_dispatch():
    return functools.partial(
        pl.kernel,
        mesh=plsc.VectorSubcoreMesh(core_axis_name="c", subcore_axis_name="s",
                                    num_cores=1),
        compiler_params=pltpu.CompilerParams(needs_layout_passes=False),
        out_type=[
            jax.ShapeDtypeStruct((N,), jnp.int32),
            jax.ShapeDtypeStruct((N,), jnp.int32),
        ],
        scratch_types=[
            pltpu.VMEM((CH1,), jnp.int32),
            pltpu.VMEM((CH1,), jnp.int32),
            pltpu.VMEM((NE,), jnp.int32),
            [pltpu.VMEM((SUB,), jnp.int32)] * NSUB1,
            [pltpu.VMEM((SUB,), jnp.int32)] * NSUB1,
            pltpu.VMEM_SHARED((N,), jnp.int32),
            pltpu.SemaphoreType.DMA,
            pltpu.SemaphoreType.DMA,
        ],
    )(_dispatch_body)


@jax.jit
def kernel(inputs, W):
    keys2, cw2, ranks2, hist2, offs2, laux = _router(inputs, W)
    keys = keys2.reshape(-1)
    ranks = ranks2.reshape(-1)
    hist = hist2.reshape(-1)
    rev, sort_ord = _build_dispatch()(keys, ranks, offs2.reshape(-1))
    return (sort_ord, rev, cw2.reshape(-1), hist, laux.reshape(()))


@functools.cache
def _build_probe():
    def _body(keys_hbm, ranks_hbm, offs_hbm, rev_hbm, sort_hbm, kv, sem):
        wid = lax.axis_index("s") * NC + lax.axis_index("c")
        base = wid * CH
        c1 = pltpu.async_copy(keys_hbm.at[pl.ds(base, CH)], kv, sem)
        c1.wait()
        c2 = pltpu.async_copy(kv, rev_hbm.at[pl.ds(base, CH)], sem)
        c3 = pltpu.async_copy(kv, sort_hbm.at[pl.ds(base, CH)], sem)
        c2.wait()
        c3.wait()

    return functools.partial(
        pl.kernel,
        mesh=plsc.VectorSubcoreMesh(core_axis_name="c", subcore_axis_name="s"),
        compiler_params=pltpu.CompilerParams(needs_layout_passes=False),
        out_type=[
            jax.ShapeDtypeStruct((N,), jnp.int32),
            jax.ShapeDtypeStruct((N,), jnp.int32),
        ],
        scratch_types=[
            pltpu.VMEM((CH,), jnp.int32),
            pltpu.SemaphoreType.DMA,
        ],
    )(_body)


# speedup vs baseline: 1.1786x; 1.1636x over previous
import functools

import jax
import jax.numpy as jnp
from jax import lax
from jax.experimental import pallas as pl
from jax.experimental.pallas import tpu as pltpu
from jax.experimental.pallas import tpu_sc as plsc

MD = 4096
NE = 64
TOKENS = 8192
KK = 2
TB = 512
NBLK = TOKENS // TB
W_LB = 0.01
W_Z = 0.001

N = TOKENS * KK
NC = 2
NS = 16
NW = NC * NS
CH = N // NW
SUB = 128
NSUB = CH // SUB


def _router_block(x_ref, w_ref, keys_ref, cw_ref, ranks_ref, hist_ref,
                  offs_ref, laux_ref, H, Psum, Cnt, Z):
    b = pl.program_id(0)

    @pl.when(b == 0)
    def _init():
        H[...] = jnp.zeros_like(H)
        Psum[...] = jnp.zeros_like(Psum)
        Cnt[...] = jnp.zeros_like(Cnt)
        Z[...] = jnp.zeros_like(Z)

    x = x_ref[...]
    w = w_ref[...]
    logits = lax.dot_general(x, w, (((1,), (1,)), ((), ())),
                             preferred_element_type=jnp.float32)
    m = jnp.max(logits, axis=1, keepdims=True)
    e = jnp.exp(logits - m)
    s = jnp.sum(e, axis=1, keepdims=True)
    probs = e / s

    li = lax.broadcasted_iota(jnp.int32, (TB, NE), 1)
    p0 = jnp.max(probs, axis=1, keepdims=True)
    i0 = jnp.min(jnp.where(probs == p0, li, NE), axis=1, keepdims=True)
    E0 = li == i0
    pm = jnp.where(E0, -1.0, probs)
    p1 = jnp.max(pm, axis=1, keepdims=True)
    i1 = jnp.min(jnp.where(pm == p1, li, NE), axis=1, keepdims=True)
    E1 = li == i1

    E0f = E0.astype(jnp.float32)
    E1f = E1.astype(jnp.float32)
    S = E0f + E1f

    r_i = lax.broadcasted_iota(jnp.int32, (TB, TB), 0)
    c_i = lax.broadcasted_iota(jnp.int32, (TB, TB), 1)
    ltri = (r_i > c_i).astype(jnp.float32)
    excl = lax.dot_general(ltri, S, (((1,), (0,)), ((), ())),
                           preferred_element_type=jnp.float32)
    Hrow = H[...]
    tot = excl + Hrow
    r0 = jnp.sum(tot * E0f, axis=1, keepdims=True)
    r1 = jnp.sum(tot * E1f, axis=1, keepdims=True)
    H[...] = Hrow + jnp.sum(S, axis=0, keepdims=True)

    ew = jnp.exp(p1 - p0)
    cw0 = 1.0 / (1.0 + ew)
    cw1 = ew * cw0

    col = lax.broadcasted_iota(jnp.int32, (TB, KK), 1)
    keys_ref[...] = jnp.where(col == 0, i0, i1)
    ranks_ref[...] = jnp.where(col == 0, r0, r1).astype(jnp.int32)
    cw_ref[...] = jnp.where(col == 0, cw0, cw1)
    hist_ref[...] = H[...].astype(jnp.int32)
    e_r = lax.broadcasted_iota(jnp.int32, (NE, NE), 0)
    e_c = lax.broadcasted_iota(jnp.int32, (NE, NE), 1)
    utri = (e_r < e_c).astype(jnp.float32)
    offs_ref[...] = lax.dot_general(H[...], utri, (((1,), (0,)), ((), ())),
                                    precision=lax.Precision.HIGHEST,
                                    preferred_element_type=jnp.float32
                                    ).astype(jnp.int32)

    Psum[...] += jnp.sum(probs, axis=0, keepdims=True)
    Cnt[...] += jnp.sum(E0f, axis=0, keepdims=True)
    lse = m + jnp.log(s)
    Z[...] += jnp.sum(lse * lse)

    @pl.when(b == NBLK - 1)
    def _finish():
        me_ce = (Psum[...] / TOKENS) * (Cnt[...] / TOKENS)
        lb = NE * jnp.sum(me_ce)
        z = Z[0, 0] / TOKENS
        laux_ref[...] = jnp.full((1, 1), W_LB * lb + W_Z * z, jnp.float32)


def _router(inputs, W):
    return pl.pallas_call(
        _router_block,
        grid=(NBLK,),
        in_specs=[
            pl.BlockSpec((TB, MD), lambda b: (b, 0)),
            pl.BlockSpec((NE, MD), lambda b: (0, 0)),
        ],
        out_specs=[
            pl.BlockSpec((TB, KK), lambda b: (b, 0)),
            pl.BlockSpec((TB, KK), lambda b: (b, 0)),
            pl.BlockSpec((TB, KK), lambda b: (b, 0)),
            pl.BlockSpec((1, NE), lambda b: (0, 0)),
            pl.BlockSpec((1, NE), lambda b: (0, 0)),
            pl.BlockSpec((1, 1), lambda b: (0, 0)),
        ],
        out_shape=[
            jax.ShapeDtypeStruct((TOKENS, KK), jnp.int32),
            jax.ShapeDtypeStruct((TOKENS, KK), jnp.float32),
            jax.ShapeDtypeStruct((TOKENS, KK), jnp.int32),
            jax.ShapeDtypeStruct((1, NE), jnp.int32),
            jax.ShapeDtypeStruct((1, NE), jnp.int32),
            jax.ShapeDtypeStruct((1, 1), jnp.float32),
        ],
        scratch_shapes=[
            pltpu.VMEM((1, NE), jnp.float32),
            pltpu.VMEM((1, NE), jnp.float32),
            pltpu.VMEM((1, NE), jnp.float32),
            pltpu.VMEM((1, 1), jnp.float32),
        ],
    )(inputs, W)


CH1 = N // NS
NSUB1 = CH1 // SUB


def _dispatch_body(keys_hbm, ranks_hbm, offs_hbm, rev_hbm, sort_hbm,
                   keys_v, ranks_v, off_v, revc, tokc, sort_sh, sem, sem2):
    sid = lax.axis_index("s")
    base = sid * CH1
    c1 = pltpu.async_copy(keys_hbm.at[pl.ds(base, CH1)], keys_v, sem)
    c2 = pltpu.async_copy(ranks_hbm.at[pl.ds(base, CH1)], ranks_v, sem)
    c3 = pltpu.async_copy(offs_hbm, off_v, sem)
    c1.wait()
    c2.wait()
    c3.wait()

    lane = lax.iota(jnp.int32, 16)
    for v in range(CH1 // 16):
        k16 = keys_v[pl.ds(v * 16, 16)]
        r16 = ranks_v[pl.ds(v * 16, 16)]
        o16 = plsc.load_gather(off_v, [k16])
        rev16 = o16 + r16
        tok16 = lax.shift_right_logical(lane + (base + v * 16), 1)
        cid, coff = divmod(v * 16, SUB)
        revc[cid][pl.ds(coff, 16)] = rev16
        tokc[cid][pl.ds(coff, 16)] = tok16

    outs = [
        pltpu.async_copy(revc[j], rev_hbm.at[pl.ds(base + j * SUB, SUB)], sem)
        for j in range(NSUB1)
    ]
    scats = [
        pltpu.async_copy(tokc[j], sort_sh.at[revc[j]], sem2)
        for j in range(NSUB1)
    ]
    for c in outs:
        c.wait()
    for c in scats:
        c.wait()
    plsc.subcore_barrier()
    pltpu.sync_copy(sort_sh.at[pl.ds(base, CH1)], keys_v)
    pltpu.sync_copy(keys_v, sort_hbm.at[pl.ds(base, CH1)])


@functools.cache
def _build_dispatch():
    return functools.partial(
        pl.kernel,
        mesh=plsc.VectorSubcoreMesh(core_axis_name="c", subcore_axis_name="s",
                                    num_cores=1),
        compiler_params=pltpu.CompilerParams(needs_layout_passes=False),
        out_type=[
            jax.ShapeDtypeStruct((N,), jnp.int32),
            jax.ShapeDtypeStruct((N,), jnp.int32),
        ],
        scratch_types=[
            pltpu.VMEM((CH1,), jnp.int32),
            pltpu.VMEM((CH1,), jnp.int32),
            pltpu.VMEM((NE,), jnp.int32),
            [pltpu.VMEM((SUB,), jnp.int32)] * NSUB1,
            [pltpu.VMEM((SUB,), jnp.int32)] * NSUB1,
            pltpu.VMEM_SHARED((N,), jnp.int32),
            pltpu.SemaphoreType.DMA,
            pltpu.SemaphoreType.DMA,
        ],
    )(_dispatch_body)


@jax.jit
def kernel(inputs, W):
    keys2, cw2, ranks2, hist2, offs2, laux = _router(inputs, W)
    keys = keys2.reshape(-1)
    ranks = ranks2.reshape(-1)
    hist = hist2.reshape(-1)
    rev, sort_ord = _build_dispatch()(keys, ranks, offs2.reshape(-1))
    return (sort_ord, rev, cw2.reshape(-1), hist, laux.reshape(()))


@functools.cache
def _build_probe():
    def _body(keys_hbm, ranks_hbm, offs_hbm, rev_hbm, sort_hbm, kv, sem):
        wid = lax.axis_index("s") * NC + lax.axis_index("c")
        base = wid * CH
        c1 = pltpu.async_copy(keys_hbm.at[pl.ds(base, CH)], kv, sem)
        c1.wait()
        c2 = pltpu.async_copy(kv, rev_hbm.at[pl.ds(base, CH)], sem)
        c3 = pltpu.async_copy(kv, sort_hbm.at[pl.ds(base, CH)], sem)
        c2.wait()
        c3.wait()

    return functools.partial(
        pl.kernel,
        mesh=plsc.VectorSubcoreMesh(core_axis_name="c", subcore_axis_name="s"),
        compiler_params=pltpu.CompilerParams(needs_layout_passes=False),
        out_type=[
            jax.ShapeDtypeStruct((N,), jnp.int32),
            jax.ShapeDtypeStruct((N,), jnp.int32),
        ],
        scratch_types=[
            pltpu.VMEM((CH,), jnp.int32),
            pltpu.SemaphoreType.DMA,
        ],
    )(_body)


# speedup vs baseline: 1.1955x; 1.0144x over previous
import functools

import jax
import jax.numpy as jnp
from jax import lax
from jax.experimental import pallas as pl
from jax.experimental.pallas import tpu as pltpu
from jax.experimental.pallas import tpu_sc as plsc

MD = 4096
NE = 64
TOKENS = 8192
KK = 2
TB = 1024
NBLK = TOKENS // TB
W_LB = 0.01
W_Z = 0.001

N = TOKENS * KK
NC = 2
NS = 16
NW = NC * NS
CH = N // NW
SUB = 128
NSUB = CH // SUB


def _router_block(x_ref, w_ref, keys_ref, cw_ref, ranks_ref, hist_ref,
                  offs_ref, laux_ref, H, Psum, Cnt, Z):
    b = pl.program_id(0)

    @pl.when(b == 0)
    def _init():
        H[...] = jnp.zeros_like(H)
        Psum[...] = jnp.zeros_like(Psum)
        Cnt[...] = jnp.zeros_like(Cnt)
        Z[...] = jnp.zeros_like(Z)

    x = x_ref[...]
    w = w_ref[...]
    logits = lax.dot_general(x, w, (((1,), (1,)), ((), ())),
                             preferred_element_type=jnp.float32)
    m = jnp.max(logits, axis=1, keepdims=True)
    e = jnp.exp(logits - m)
    s = jnp.sum(e, axis=1, keepdims=True)
    probs = e / s

    li = lax.broadcasted_iota(jnp.int32, (TB, NE), 1)
    p0 = jnp.max(probs, axis=1, keepdims=True)
    i0 = jnp.min(jnp.where(probs == p0, li, NE), axis=1, keepdims=True)
    E0 = li == i0
    pm = jnp.where(E0, -1.0, probs)
    p1 = jnp.max(pm, axis=1, keepdims=True)
    i1 = jnp.min(jnp.where(pm == p1, li, NE), axis=1, keepdims=True)
    E1 = li == i1

    E0f = E0.astype(jnp.float32)
    E1f = E1.astype(jnp.float32)
    S = E0f + E1f

    r_i = lax.broadcasted_iota(jnp.int32, (TB, TB), 0)
    c_i = lax.broadcasted_iota(jnp.int32, (TB, TB), 1)
    ltri = (r_i > c_i).astype(jnp.float32)
    excl = lax.dot_general(ltri, S, (((1,), (0,)), ((), ())),
                           preferred_element_type=jnp.float32)
    Hrow = H[...]
    tot = excl + Hrow
    r0 = jnp.sum(tot * E0f, axis=1, keepdims=True)
    r1 = jnp.sum(tot * E1f, axis=1, keepdims=True)
    H[...] = Hrow + jnp.sum(S, axis=0, keepdims=True)

    ew = jnp.exp(p1 - p0)
    cw0 = 1.0 / (1.0 + ew)
    cw1 = ew * cw0

    col = lax.broadcasted_iota(jnp.int32, (TB, KK), 1)
    keys_ref[...] = jnp.where(col == 0, i0, i1)
    ranks_ref[...] = jnp.where(col == 0, r0, r1).astype(jnp.int32)
    cw_ref[...] = jnp.where(col == 0, cw0, cw1)
    hist_ref[...] = H[...].astype(jnp.int32)
    e_r = lax.broadcasted_iota(jnp.int32, (NE, NE), 0)
    e_c = lax.broadcasted_iota(jnp.int32, (NE, NE), 1)
    utri = (e_r < e_c).astype(jnp.float32)
    offs_ref[...] = lax.dot_general(H[...], utri, (((1,), (0,)), ((), ())),
                                    precision=lax.Precision.HIGHEST,
                                    preferred_element_type=jnp.float32
                                    ).astype(jnp.int32)

    Psum[...] += jnp.sum(probs, axis=0, keepdims=True)
    Cnt[...] += jnp.sum(E0f, axis=0, keepdims=True)
    lse = m + jnp.log(s)
    Z[...] += jnp.sum(lse * lse)

    @pl.when(b == NBLK - 1)
    def _finish():
        me_ce = (Psum[...] / TOKENS) * (Cnt[...] / TOKENS)
        lb = NE * jnp.sum(me_ce)
        z = Z[0, 0] / TOKENS
        laux_ref[...] = jnp.full((1, 1), W_LB * lb + W_Z * z, jnp.float32)


def _router(inputs, W):
    return pl.pallas_call(
        _router_block,
        grid=(NBLK,),
        in_specs=[
            pl.BlockSpec((TB, MD), lambda b: (b, 0)),
            pl.BlockSpec((NE, MD), lambda b: (0, 0)),
        ],
        out_specs=[
            pl.BlockSpec((TB, KK), lambda b: (b, 0)),
            pl.BlockSpec((TB, KK), lambda b: (b, 0)),
            pl.BlockSpec((TB, KK), lambda b: (b, 0)),
            pl.BlockSpec((1, NE), lambda b: (0, 0)),
            pl.BlockSpec((1, NE), lambda b: (0, 0)),
            pl.BlockSpec((1, 1), lambda b: (0, 0)),
        ],
        out_shape=[
            jax.ShapeDtypeStruct((TOKENS, KK), jnp.int32),
            jax.ShapeDtypeStruct((TOKENS, KK), jnp.float32),
            jax.ShapeDtypeStruct((TOKENS, KK), jnp.int32),
            jax.ShapeDtypeStruct((1, NE), jnp.int32),
            jax.ShapeDtypeStruct((1, NE), jnp.int32),
            jax.ShapeDtypeStruct((1, 1), jnp.float32),
        ],
        scratch_shapes=[
            pltpu.VMEM((1, NE), jnp.float32),
            pltpu.VMEM((1, NE), jnp.float32),
            pltpu.VMEM((1, NE), jnp.float32),
            pltpu.VMEM((1, 1), jnp.float32),
        ],
    )(inputs, W)


CH1 = N // NS
NSUB1 = CH1 // SUB


def _dispatch_body(keys_hbm, ranks_hbm, offs_hbm, rev_hbm, sort_hbm,
                   keys_v, ranks_v, off_v, revc, tokc, sort_sh, sem, sem2):
    sid = lax.axis_index("s")
    base = sid * CH1
    c1 = pltpu.async_copy(keys_hbm.at[pl.ds(base, CH1)], keys_v, sem)
    c2 = pltpu.async_copy(ranks_hbm.at[pl.ds(base, CH1)], ranks_v, sem)
    c3 = pltpu.async_copy(offs_hbm, off_v, sem)
    c1.wait()
    c2.wait()
    c3.wait()

    lane = lax.iota(jnp.int32, 16)
    for v in range(CH1 // 16):
        k16 = keys_v[pl.ds(v * 16, 16)]
        r16 = ranks_v[pl.ds(v * 16, 16)]
        o16 = plsc.load_gather(off_v, [k16])
        rev16 = o16 + r16
        tok16 = lax.shift_right_logical(lane + (base + v * 16), 1)
        cid, coff = divmod(v * 16, SUB)
        revc[cid][pl.ds(coff, 16)] = rev16
        tokc[cid][pl.ds(coff, 16)] = tok16

    outs = [
        pltpu.async_copy(revc[j], rev_hbm.at[pl.ds(base + j * SUB, SUB)], sem)
        for j in range(NSUB1)
    ]
    scats = [
        pltpu.async_copy(tokc[j], sort_sh.at[revc[j]], sem2)
        for j in range(NSUB1)
    ]
    for c in outs:
        c.wait()
    for c in scats:
        c.wait()
    plsc.subcore_barrier()
    pltpu.sync_copy(sort_sh.at[pl.ds(base, CH1)], keys_v)
    pltpu.sync_copy(keys_v, sort_hbm.at[pl.ds(base, CH1)])


@functools.cache
def _build_dispatch():
    return functools.partial(
        pl.kernel,
        mesh=plsc.VectorSubcoreMesh(core_axis_name="c", subcore_axis_name="s",
                                    num_cores=1),
        compiler_params=pltpu.CompilerParams(needs_layout_passes=False),
        out_type=[
            jax.ShapeDtypeStruct((N,), jnp.int32),
            jax.ShapeDtypeStruct((N,), jnp.int32),
        ],
        scratch_types=[
            pltpu.VMEM((CH1,), jnp.int32),
            pltpu.VMEM((CH1,), jnp.int32),
            pltpu.VMEM((NE,), jnp.int32),
            [pltpu.VMEM((SUB,), jnp.int32)] * NSUB1,
            [pltpu.VMEM((SUB,), jnp.int32)] * NSUB1,
            pltpu.VMEM_SHARED((N,), jnp.int32),
            pltpu.SemaphoreType.DMA,
            pltpu.SemaphoreType.DMA,
        ],
    )(_dispatch_body)


@jax.jit
def kernel(inputs, W):
    keys2, cw2, ranks2, hist2, offs2, laux = _router(inputs, W)
    keys = keys2.reshape(-1)
    ranks = ranks2.reshape(-1)
    hist = hist2.reshape(-1)
    rev, sort_ord = _build_dispatch()(keys, ranks, offs2.reshape(-1))
    return (sort_ord, rev, cw2.reshape(-1), hist, laux.reshape(()))


@functools.cache
def _build_probe():
    def _body(keys_hbm, ranks_hbm, offs_hbm, rev_hbm, sort_hbm, kv, sem):
        wid = lax.axis_index("s") * NC + lax.axis_index("c")
        base = wid * CH
        c1 = pltpu.async_copy(keys_hbm.at[pl.ds(base, CH)], kv, sem)
        c1.wait()
        c2 = pltpu.async_copy(kv, rev_hbm.at[pl.ds(base, CH)], sem)
        c3 = pltpu.async_copy(kv, sort_hbm.at[pl.ds(base, CH)], sem)
        c2.wait()
        c3.wait()

    return functools.partial(
        pl.kernel,
        mesh=plsc.VectorSubcoreMesh(core_axis_name="c", subcore_axis_name="s"),
        compiler_params=pltpu.CompilerParams(needs_layout_passes=False),
        out_type=[
            jax.ShapeDtypeStruct((N,), jnp.int32),
            jax.ShapeDtypeStruct((N,), jnp.int32),
        ],
        scratch_types=[
            pltpu.VMEM((CH,), jnp.int32),
            pltpu.SemaphoreType.DMA,
        ],
    )(_body)
